# baseline (speedup 1.0000x reference)
import functools

import jax
import jax.numpy as jnp
from jax import lax
from jax.experimental import pallas as pl
from jax.experimental.pallas import tpu as pltpu
from jax.experimental.pallas import tpu_sc as plsc

HIDDEN = 32
NC = 2
NS = 16
NW = NC * NS
NB = 16384
SEQ = 26
B = NB * SEQ
BT = NB // 128
TPW = BT // NW
B_PER_W = B // NW
TBS = 2
ROWS = 128 * TBS
STEPS = (TPW // TBS) * SEQ
RING = 4
TABLE = 1000000
HALF = TABLE // 2

_mesh = plsc.VectorSubcoreMesh(core_axis_name="c", subcore_axis_name="s")

_A_SCRATCH = [
    pltpu.VMEM((B_PER_W,), jnp.int32),
    pltpu.VMEM((B_PER_W,), jnp.int32),
    [pltpu.VMEM((ROWS, HIDDEN), jnp.float32) for _ in range(RING)],
    [pltpu.VMEM((HIDDEN // 8, TBS, 8, 128), jnp.float32) for _ in range(RING)],
    [pltpu.SemaphoreType.DMA for _ in range(RING)],
    [pltpu.SemaphoreType.DMA for _ in range(RING)],
]
_B_SCRATCH = _A_SCRATCH + [
    [pltpu.VMEM((HIDDEN // 8, TBS, 8, 128), jnp.float32) for _ in range(RING)],
    [pltpu.SemaphoreType.DMA for _ in range(RING)],
]


def _make(merge):
    def body(*refs):
        if merge:
            (idx_hbm, w_hbm, ina_hbm, out_hbm, idx2, idx2r, rows, shuf,
             gsem, ssem, inb, isem) = refs
        else:
            (idx_hbm, w_hbm, out_hbm, idx2, idx2r, rows, shuf,
             gsem, ssem) = refs
        wid = lax.axis_index("s") * NC + lax.axis_index("c")
        base = wid * B_PER_W
        iota = lax.iota(jnp.int32, 16)

        def perm_load(g, carry):
            tp = g // (SEQ * 16)
            rem = g % (SEQ * 16)
            s = rem // 16
            b0 = (rem % 16) * 16
            src = SEQ * (tp * ROWS + b0) + s + SEQ * iota
            v = plsc.load_gather(idx2r, [src])
            idx2[pl.ds((tp * SEQ + s) * ROWS + b0, 16)] = v
            return carry

        def clamp_body(q, carry):
            v = idx2[pl.ds(q * 16, 16)]
            if merge:
                idx2r[pl.ds(q * 16, 16)] = v
                idx2[pl.ds(q * 16, 16)] = jnp.minimum(
                    jnp.maximum(v - HALF, 0), HALF - 1)
            else:
                idx2[pl.ds(q * 16, 16)] = jnp.minimum(v, HALF - 1)
            return carry

        def stage_and_permute():
            pltpu.sync_copy(idx_hbm.at[pl.ds(base, B_PER_W)], idx2r)
            lax.fori_loop(0, STEPS * 16, perm_load, 0)
            lax.fori_loop(0, B_PER_W // 16, clamp_body, 0)

        def start_gather(k, par):
            pltpu.make_async_copy(
                w_hbm.at[idx2.at[pl.ds(k * ROWS, ROWS)]], rows[par], gsem[par]
            ).start()

        def wait_gather(par):
            pltpu.make_async_copy(
                w_hbm.at[idx2.at[pl.ds(0, ROWS)]], rows[par], gsem[par]).wait()

        def blk(k):
            tp = k // SEQ
            s = k % SEQ
            return s, wid * TPW + tp * TBS

        def store_block(k, par):
            s, bt0 = blk(k)
            pltpu.make_async_copy(
                shuf[par], out_hbm.at[s, :, pl.ds(bt0, TBS)], ssem[par]).start()

        def wait_store(par):
            pltpu.make_async_copy(
                out_hbm.at[0, :, pl.ds(0, TBS)], shuf[par], ssem[par]).wait()

        def start_inread(k, par):
            s, bt0 = blk(k)
            pltpu.make_async_copy(
                ina_hbm.at[s, :, pl.ds(bt0, TBS)], inb[par], isem[par]).start()

        def wait_inread(par):
            pltpu.make_async_copy(
                ina_hbm.at[0, :, pl.ds(0, TBS)], inb[par], isem[par]).wait()

        def shuffle(k, par):
            kbase = k * ROWS

            def shuf_body(h, carry):
                ht = h // 8
                hs = h % 8
                hvec = jnp.full((16,), h, jnp.int32)
                for tb in range(TBS):
                    for g in range(8):
                        b0 = tb * 128 + g * 16
                        v = plsc.load_gather(rows[par], [b0 + iota, hvec])
                        if merge:
                            va = inb[par][ht, tb, hs, pl.ds(g * 16, 16)]
                            im = idx2r[pl.ds(kbase + b0, 16)]
                            v = jnp.where(im < HALF, va, v)
                        shuf[par][ht, tb, hs, pl.ds(g * 16, 16)] = v
                return carry

            lax.fori_loop(0, HIDDEN, shuf_body, 0)

        def step(k, par):
            wait_gather(par)
            if merge:
                wait_inread(par)

            @pl.when(k >= RING)
            def _():
                wait_store(par)

            shuffle(k, par)

            @pl.when(k + RING < STEPS)
            def _():
                start_gather(k + RING, par)
                if merge:
                    start_inread(k + RING, par)

            store_block(k, par)

        def loop_body(i, carry):
            for p in range(RING):
                step(RING * i + p, p)
            return carry

        stage_and_permute()
        for p in range(RING):
            start_gather(p, p)
            if merge:
                start_inread(p, p)
        lax.fori_loop(0, STEPS // RING, loop_body, 0)
        for p in range(RING):
            wait_store(p)

    scratch = list(_B_SCRATCH) if merge else list(_A_SCRATCH)
    return functools.partial(
        pl.kernel,
        mesh=_mesh,
        out_type=jax.ShapeDtypeStruct(
            (SEQ, HIDDEN // 8, BT, 8, 128), jnp.float32),
        scratch_types=scratch,
        compiler_params=pltpu.CompilerParams(
            use_tc_tiling_on_sc=False, needs_layout_passes=False),
    )(body)


_kernel_a = _make(False)
_kernel_b = _make(True)


@jax.jit
def kernel(x, weight):
    flat = x.reshape(-1).astype(jnp.int32)
    wa = weight[:HALF]
    wb = weight[HALF:]
    out_a = _kernel_a(flat, wa)
    out_b = _kernel_b(flat, wb, out_a)
    return out_b.transpose((2, 4, 0, 1, 3)).reshape(NB, SEQ, HIDDEN)


# speedup vs baseline: 6.3755x; 6.3755x over previous
import functools

import jax
import jax.numpy as jnp
from jax import lax
from jax.experimental import pallas as pl
from jax.experimental.pallas import tpu as pltpu
from jax.experimental.pallas import tpu_sc as plsc

HIDDEN = 32
NC = 2
NS = 16
NW = NC * NS
B = 16384 * 26
B_PER_W = B // NW
CHUNK = 416
N_CHUNKS = B_PER_W // CHUNK
NBUF = 8
LOOKAHEAD = 6

_mesh = plsc.VectorSubcoreMesh(core_axis_name="c", subcore_axis_name="s")


@functools.partial(
    pl.kernel,
    mesh=_mesh,
    out_type=jax.ShapeDtypeStruct((B, HIDDEN), jnp.float32),
    scratch_types=[
        pltpu.VMEM((B_PER_W,), jnp.int32),
        [pltpu.VMEM((CHUNK, HIDDEN), jnp.float32) for _ in range(NBUF)],
        [pltpu.SemaphoreType.DMA for _ in range(NBUF)],
        [pltpu.SemaphoreType.DMA for _ in range(NBUF)],
    ],
    compiler_params=pltpu.CompilerParams(use_tc_tiling_on_sc=False),
)
def _gather_all(idx_hbm, w_hbm, out_hbm, idx_v, rows, gsem, ssem):
    wid = lax.axis_index("s") * NC + lax.axis_index("c")
    base = wid * B_PER_W

    pltpu.sync_copy(idx_hbm.at[pl.ds(base, B_PER_W)], idx_v)

    def start_gather(c):
        b = c % NBUF
        return pltpu.async_copy(
            w_hbm.at[idx_v.at[pl.ds(c * CHUNK, CHUNK)]], rows[b], gsem[b])

    gathers = [None] * N_CHUNKS
    stores = [None] * N_CHUNKS
    for c in range(LOOKAHEAD):
        gathers[c] = start_gather(c)
    for c in range(N_CHUNKS):
        b = c % NBUF
        nxt = c + LOOKAHEAD
        if nxt < N_CHUNKS:
            if nxt >= NBUF:
                stores[nxt - NBUF].wait()
            gathers[nxt] = start_gather(nxt)
        gathers[c].wait()
        stores[c] = pltpu.async_copy(
            rows[b], out_hbm.at[pl.ds(base + c * CHUNK, CHUNK)], ssem[b])
    for c in range(max(0, N_CHUNKS - NBUF), N_CHUNKS):
        if stores[c] is not None:
            stores[c].wait()


@jax.jit
def kernel(x, weight):
    flat = x.reshape(-1).astype(jnp.int32)
    out = _gather_all(flat, weight)
    return out.reshape(x.shape + (weight.shape[1],))
